# baseline (speedup 1.0000x reference)
import functools

import jax
import jax.numpy as jnp
from jax import lax
from jax.experimental import pallas as pl
from jax.experimental.pallas import tpu as pltpu
from jax.experimental.pallas import tpu_sc as plsc

L = 2048
D_IN = 1024
D_MDL = 1024
H = 16
DH = D_MDL // H
E = 65536
TEMP = 0.125


_ROWS = 256
_NT = L // _ROWS

def _split3(x):
    hi = x.astype(jnp.bfloat16)
    lo = (x - hi.astype(jnp.float32)).astype(jnp.bfloat16)
    return hi, lo


def _dot3(a, b, dims):
    ah, al = _split3(a)
    bh, bl = _split3(b)
    d = lambda x, y: lax.dot_general(x, y, dims,
                                     preferred_element_type=jnp.float32)
    return d(ah, bh) + d(ah, bl) + d(al, bh)


_MM_DIMS = (((1,), (0,)), ((), ()))
_QK_DIMS = (((1,), (1,)), ((), ()))


def _proj_body(xq, xk, xv, wq, wk, wv, bq, bk, bv,
               oqh, oql, okh, okl, ov):
    lq = (_dot3(xq[...], wq[...], _MM_DIMS) + bq[...]) * TEMP
    qh = lq.astype(jnp.bfloat16)
    ql = (lq - qh.astype(jnp.float32)).astype(jnp.bfloat16)
    lk = _dot3(xk[...], wk[...], _MM_DIMS) + bk[...]
    kh = lk.astype(jnp.bfloat16)
    kl = (lk - kh.astype(jnp.float32)).astype(jnp.bfloat16)
    vb = (_dot3(xv[...], wv[...], _MM_DIMS) + bv[...]).astype(jnp.bfloat16)
    zcol = jnp.zeros((_ROWS, 128 - DH - 1), jnp.bfloat16)
    onecol = jnp.ones((_ROWS, 1), jnp.bfloat16)
    for h in range(H):
        lo, hi = h * DH, (h + 1) * DH
        oqh[h] = qh[:, lo:hi]
        oql[h] = ql[:, lo:hi]
        okh[h] = kh[:, lo:hi]
        okl[h] = kl[:, lo:hi]
        ov[h] = jnp.concatenate([vb[:, lo:hi], onecol, zcol], axis=-1)


def _project(q2, k2, v2, Wq, Wk, Wv, bq, bk, bv):
    x_spec = pl.BlockSpec((_ROWS, D_IN), lambda i: (i, 0))
    w_spec = pl.BlockSpec((D_IN, D_MDL), lambda i: (0, 0))
    b_spec = pl.BlockSpec((1, D_MDL), lambda i: (0, 0))
    o_spec = pl.BlockSpec((H, _ROWS, DH), lambda i: (0, i, 0))
    o_sd = jax.ShapeDtypeStruct((H, L, DH), jnp.bfloat16)
    va_spec = pl.BlockSpec((H, _ROWS, 128), lambda i: (0, i, 0))
    va_sd = jax.ShapeDtypeStruct((H, L, 128), jnp.bfloat16)
    return pl.pallas_call(
        _proj_body,
        grid=(_NT,),
        in_specs=[x_spec, x_spec, x_spec, w_spec, w_spec, w_spec,
                  b_spec, b_spec, b_spec],
        out_specs=[o_spec, o_spec, o_spec, o_spec, va_spec],
        out_shape=[o_sd, o_sd, o_sd, o_sd, va_sd],
    )(q2, k2, v2, Wq, Wk, Wv, bq, bk, bv)


_NS = 16
_EPW = E // _NS
_NCHUNK = _EPW // 128
_QROWS = 512
_QWORDS = _QROWS * L
_DUMP = _QWORDS
_ZW = 8192
_SLICE = _QWORDS // _NS


def _build_w_body(a0_hbm, a1_hbm, epe_hbm, out_hbm,
                  a0_v, a1_v, val_v, idx_v, zeros_v, flush_v, w_sh, sem):
    c = lax.axis_index("c")
    s = lax.axis_index("s")
    base_e = s * _EPW

    cp0 = pltpu.async_copy(a0_hbm.at[pl.ds(base_e, _EPW)], a0_v, sem)
    cp1 = pltpu.async_copy(a1_hbm.at[pl.ds(base_e, _EPW)], a1_v, sem)
    cp2 = pltpu.async_copy(epe_hbm.at[pl.ds(base_e, _EPW)], val_v, sem)

    def _zero_step(t, carry):
        zeros_v[pl.ds(t * 16, 16)] = jnp.zeros((16,), jnp.float32)
        return carry

    lax.fori_loop(0, _ZW // 16, _zero_step, 0)
    cp0.wait()
    cp1.wait()
    cp2.wait()

    def _val_step(t, carry):
        x = val_v[pl.ds(t * 16, 16)]
        val_v[pl.ds(t * 16, 16)] = jnp.exp(x * TEMP)
        return carry

    lax.fori_loop(0, _EPW // 16, _val_step, 0)

    for p in range(2):
        base_row = c * (2 * _QROWS) + p * _QROWS

        zcps = [pltpu.async_copy(
                    zeros_v, w_sh.at[pl.ds(s * _SLICE + z * _ZW, _ZW)], sem)
                for z in range(_SLICE // _ZW)]
        def _idx_step(t, carry):
            a0 = a0_v[pl.ds(t * 16, 16)]
            a1 = a1_v[pl.ds(t * 16, 16)]
            rel = a0 - base_row
            ok = (rel >= 0) & (rel < _QROWS)
            idx_v[t // 8, pl.ds((t % 8) * 16, 16)] = jnp.where(
                ok, rel * L + a1, _DUMP)
            return carry

        lax.fori_loop(0, _EPW // 16, _idx_step, 0)
        def _dump_step(t, carry):
            idx_v[_NCHUNK, pl.ds(t * 16, 16)] = jnp.full(
                (16,), _DUMP, jnp.int32)
            return carry

        lax.fori_loop(0, 8, _dump_step, 0)
        for cp in zcps:
            cp.wait()
        plsc.subcore_barrier()

        scps = [pltpu.async_copy(val_v.at[pl.ds(j * 128, 128)],
                                 w_sh.at[idx_v.at[j]], sem, add=True)
                for j in range(_NCHUNK)]
        scps += [pltpu.async_copy(zeros_v.at[pl.ds(0, 128)],
                                  w_sh.at[idx_v.at[_NCHUNK]], sem, add=True)
                 for _ in range(2)]
        for cp in scps:
            cp.wait()
        pltpu.sync_copy(zeros_v.at[pl.ds(0, 16)],
                        w_sh.at[pl.ds(_QWORDS, 16)])
        pltpu.sync_copy(w_sh.at[pl.ds(_QWORDS, 16)], flush_v)
        plsc.subcore_barrier()
        out_base = base_row * L + s * _SLICE
        pltpu.sync_copy(w_sh.at[pl.ds(s * _SLICE, _SLICE)],
                        out_hbm.at[pl.ds(out_base, _SLICE)])
        plsc.subcore_barrier()


def _build_w(a0, a1, epe):
    mesh = plsc.VectorSubcoreMesh(core_axis_name="c", subcore_axis_name="s")
    return pl.kernel(
        _build_w_body,
        out_type=jax.ShapeDtypeStruct((L * L,), jnp.float32),
        mesh=mesh,
        scratch_types=[
            pltpu.VMEM((_EPW,), jnp.int32),
            pltpu.VMEM((_EPW,), jnp.int32),
            pltpu.VMEM((_EPW,), jnp.float32),
            pltpu.VMEM((_NCHUNK + 1, 128), jnp.int32),
            pltpu.VMEM((_ZW,), jnp.float32),
            pltpu.VMEM((16,), jnp.float32),
            pltpu.VMEM_SHARED((_QWORDS + 16,), jnp.float32),
            pltpu.SemaphoreType.DMA,
        ],
    )(a0, a1, epe)


def _attn_body(qh, ql, kh, kl, w_ref, vaug, wo3, bo_ref, out_ref):
    h = pl.program_id(1)
    d = lambda a, b: lax.dot_general(a, b, _QK_DIMS,
                                     preferred_element_type=jnp.float32)
    s = d(qh[0], kh[h]) + d(qh[0], kl[h]) + d(ql[0], kh[h])
    p = (w_ref[...] * jnp.exp(s)).astype(jnp.bfloat16)
    pv = lax.dot_general(p, vaug[h], _MM_DIMS,
                         preferred_element_type=jnp.float32)
    den = pv[:, DH:DH + 1] + 1e-16
    ho = pv[:, :DH] / den
    contrib = lax.dot_general(ho.astype(jnp.bfloat16), wo3[0], _MM_DIMS,
                              preferred_element_type=jnp.float32)

    @pl.when(h == 0)
    def _():
        out_ref[...] = contrib + bo_ref[...]

    @pl.when(h > 0)
    def _():
        out_ref[...] += contrib


def _attention(qh3, ql3, kh3, kl3, wmat, vaug, wo3, bo):
    return pl.pallas_call(
        _attn_body,
        grid=(_NT, H),
        in_specs=[
            pl.BlockSpec((1, _ROWS, DH), lambda i, h: (h, i, 0)),
            pl.BlockSpec((1, _ROWS, DH), lambda i, h: (h, i, 0)),
            pl.BlockSpec((H, L, DH), lambda i, h: (0, 0, 0)),
            pl.BlockSpec((H, L, DH), lambda i, h: (0, 0, 0)),
            pl.BlockSpec((_ROWS, L), lambda i, h: (i, 0)),
            pl.BlockSpec((H, L, 128), lambda i, h: (0, 0, 0)),
            pl.BlockSpec((1, DH, D_MDL), lambda i, h: (h, 0, 0)),
            pl.BlockSpec((1, D_MDL), lambda i, h: (0, 0)),
        ],
        out_specs=pl.BlockSpec((_ROWS, D_MDL), lambda i, h: (i, 0)),
        out_shape=jax.ShapeDtypeStruct((L, D_MDL), jnp.float32),
    )(qh3, ql3, kh3, kl3, wmat, vaug, wo3, bo)


def kernel(queries, keys, values, adj, edge_pos_enc, Wq, bq, Wk, bk, Wv, bv,
           Wo, bo):
    q2 = queries.reshape(L, D_IN)
    k2 = keys.reshape(L, D_IN)
    v2 = values.reshape(L, D_IN)

    qh3, ql3, kh3, kl3, vaug = _project(q2, k2, v2, Wq, Wk, Wv,
                                        bq.reshape(1, D_MDL),
                                        bk.reshape(1, D_MDL),
                                        bv.reshape(1, D_MDL))

    wflat = _build_w(adj[0], adj[1], edge_pos_enc)
    wmat = wflat.reshape(L, L)
    wo3 = Wo.astype(jnp.bfloat16).reshape(H, DH, D_MDL)

    out = _attention(qh3, ql3, kh3, kl3, wmat, vaug, wo3,
                     bo.reshape(1, D_MDL))
    return out.reshape(1, L, D_MDL)


# speedup vs baseline: 1.2670x; 1.2670x over previous
import functools

import jax
import jax.numpy as jnp
from jax import lax
from jax.experimental import pallas as pl
from jax.experimental.pallas import tpu as pltpu
from jax.experimental.pallas import tpu_sc as plsc

L = 2048
D_IN = 1024
D_MDL = 1024
H = 16
DH = D_MDL // H
E = 65536
TEMP = 0.125


_ROWS = 256
_NT = L // _ROWS

def _split3(x):
    hi = x.astype(jnp.bfloat16)
    lo = (x - hi.astype(jnp.float32)).astype(jnp.bfloat16)
    return hi, lo


def _dot3(a, b, dims):
    ah, al = _split3(a)
    bh, bl = _split3(b)
    d = lambda x, y: lax.dot_general(x, y, dims,
                                     preferred_element_type=jnp.float32)
    return d(ah, bh) + d(ah, bl) + d(al, bh)


_MM_DIMS = (((1,), (0,)), ((), ()))
_QK_DIMS = (((1,), (1,)), ((), ()))


def _proj_body(xq, xk, xv, wq, wk, wv, bq, bk, bv, oqh, okh, ov):
    lq = (_dot3(xq[...], wq[...], _MM_DIMS) + bq[...]) * TEMP
    qh = lq.astype(jnp.bfloat16)
    lk = _dot3(xk[...], wk[...], _MM_DIMS) + bk[...]
    kh = lk.astype(jnp.bfloat16)
    vb = (_dot3(xv[...], wv[...], _MM_DIMS) + bv[...]).astype(jnp.bfloat16)
    zcol = jnp.zeros((_ROWS, 128 - DH - 1), jnp.bfloat16)
    onecol = jnp.ones((_ROWS, 1), jnp.bfloat16)
    for h in range(H):
        lo, hi = h * DH, (h + 1) * DH
        oqh[h] = qh[:, lo:hi]
        okh[h] = kh[:, lo:hi]
        ov[h] = jnp.concatenate([vb[:, lo:hi], onecol, zcol], axis=-1)


def _project(q2, k2, v2, Wq, Wk, Wv, bq, bk, bv):
    x_spec = pl.BlockSpec((_ROWS, D_IN), lambda i: (i, 0))
    w_spec = pl.BlockSpec((D_IN, D_MDL), lambda i: (0, 0))
    b_spec = pl.BlockSpec((1, D_MDL), lambda i: (0, 0))
    o_spec = pl.BlockSpec((H, _ROWS, DH), lambda i: (0, i, 0))
    o_sd = jax.ShapeDtypeStruct((H, L, DH), jnp.bfloat16)
    va_spec = pl.BlockSpec((H, _ROWS, 128), lambda i: (0, i, 0))
    va_sd = jax.ShapeDtypeStruct((H, L, 128), jnp.bfloat16)
    return pl.pallas_call(
        _proj_body,
        grid=(_NT,),
        in_specs=[x_spec, x_spec, x_spec, w_spec, w_spec, w_spec,
                  b_spec, b_spec, b_spec],
        out_specs=[o_spec, o_spec, va_spec],
        out_shape=[o_sd, o_sd, va_sd],
    )(q2, k2, v2, Wq, Wk, Wv, bq, bk, bv)


_NS = 16
_EPW = E // _NS
_NCHUNK = _EPW // 128
_QROWS = 512
_QWORDS = _QROWS * L
_DUMP = _QWORDS
_ZW = 8192
_SLICE = _QWORDS // _NS


def _build_w_body(a0_hbm, a1_hbm, epe_hbm, out_hbm,
                  a0_v, a1_v, val_v, idx_v, zeros_v, flush_v, w_sh, sem):
    c = lax.axis_index("c")
    s = lax.axis_index("s")
    base_e = s * _EPW

    cp0 = pltpu.async_copy(a0_hbm.at[pl.ds(base_e, _EPW)], a0_v, sem)
    cp1 = pltpu.async_copy(a1_hbm.at[pl.ds(base_e, _EPW)], a1_v, sem)
    cp2 = pltpu.async_copy(epe_hbm.at[pl.ds(base_e, _EPW)], val_v, sem)

    def _zero_step(t, carry):
        zeros_v[pl.ds(t * 16, 16)] = jnp.zeros((16,), jnp.float32)
        return carry

    lax.fori_loop(0, _ZW // 16, _zero_step, 0)
    cp0.wait()
    cp1.wait()
    cp2.wait()

    def _val_step(t, carry):
        x = val_v[pl.ds(t * 16, 16)]
        val_v[pl.ds(t * 16, 16)] = jnp.exp(x * TEMP)
        return carry

    lax.fori_loop(0, _EPW // 16, _val_step, 0)

    for p in range(2):
        base_row = c * (2 * _QROWS) + p * _QROWS

        zcps = [pltpu.async_copy(
                    zeros_v, w_sh.at[pl.ds(s * _SLICE + z * _ZW, _ZW)], sem)
                for z in range(_SLICE // _ZW)]
        def _idx_step(t, carry):
            a0 = a0_v[pl.ds(t * 16, 16)]
            a1 = a1_v[pl.ds(t * 16, 16)]
            rel = a0 - base_row
            ok = (rel >= 0) & (rel < _QROWS)
            idx_v[t // 8, pl.ds((t % 8) * 16, 16)] = jnp.where(
                ok, rel * L + a1, _DUMP)
            return carry

        lax.fori_loop(0, _EPW // 16, _idx_step, 0)
        def _dump_step(t, carry):
            idx_v[_NCHUNK, pl.ds(t * 16, 16)] = jnp.full(
                (16,), _DUMP, jnp.int32)
            return carry

        lax.fori_loop(0, 8, _dump_step, 0)
        for cp in zcps:
            cp.wait()
        plsc.subcore_barrier()

        scps = [pltpu.async_copy(val_v.at[pl.ds(j * 128, 128)],
                                 w_sh.at[idx_v.at[j]], sem, add=True)
                for j in range(_NCHUNK)]
        scps += [pltpu.async_copy(zeros_v.at[pl.ds(0, 128)],
                                  w_sh.at[idx_v.at[_NCHUNK]], sem, add=True)
                 for _ in range(2)]
        for cp in scps:
            cp.wait()
        pltpu.sync_copy(zeros_v.at[pl.ds(0, 16)],
                        w_sh.at[pl.ds(_QWORDS, 16)])
        pltpu.sync_copy(w_sh.at[pl.ds(_QWORDS, 16)], flush_v)
        plsc.subcore_barrier()
        row0 = base_row + s * (_SLICE // L)
        ocps = [pltpu.async_copy(
                    w_sh.at[pl.ds(s * _SLICE + r * L, L)],
                    out_hbm.at[row0 + r], sem)
                for r in range(_SLICE // L)]
        for cp in ocps:
            cp.wait()
        plsc.subcore_barrier()


def _build_w(a0, a1, epe):
    mesh = plsc.VectorSubcoreMesh(core_axis_name="c", subcore_axis_name="s")
    return pl.kernel(
        _build_w_body,
        out_type=jax.ShapeDtypeStruct((L, L), jnp.float32),
        mesh=mesh,
        scratch_types=[
            pltpu.VMEM((_EPW,), jnp.int32),
            pltpu.VMEM((_EPW,), jnp.int32),
            pltpu.VMEM((_EPW,), jnp.float32),
            pltpu.VMEM((_NCHUNK + 1, 128), jnp.int32),
            pltpu.VMEM((_ZW,), jnp.float32),
            pltpu.VMEM((16,), jnp.float32),
            pltpu.VMEM_SHARED((_QWORDS + 16,), jnp.float32),
            pltpu.SemaphoreType.DMA,
        ],
    )(a0, a1, epe)


def _attn_body(qh, kh, w_ref, vaug, wo3, bo_ref, out_ref):
    h = pl.program_id(1)
    s = lax.dot_general(qh[0], kh[h], _QK_DIMS,
                        preferred_element_type=jnp.float32)
    p = (w_ref[...] * jnp.exp(s)).astype(jnp.bfloat16)
    pv = lax.dot_general(p, vaug[h], _MM_DIMS,
                         preferred_element_type=jnp.float32)
    den = pv[:, DH:DH + 1] + 1e-16
    ho = pv[:, :DH] / den
    contrib = lax.dot_general(ho.astype(jnp.bfloat16), wo3[0], _MM_DIMS,
                              preferred_element_type=jnp.float32)

    @pl.when(h == 0)
    def _():
        out_ref[...] = contrib + bo_ref[...]

    @pl.when(h > 0)
    def _():
        out_ref[...] += contrib


def _attention(qh3, kh3, wmat, vaug, wo3, bo):
    return pl.pallas_call(
        _attn_body,
        grid=(_NT, H),
        in_specs=[
            pl.BlockSpec((1, _ROWS, DH), lambda i, h: (h, i, 0)),
            pl.BlockSpec((H, L, DH), lambda i, h: (0, 0, 0)),
            pl.BlockSpec((_ROWS, L), lambda i, h: (i, 0)),
            pl.BlockSpec((H, L, 128), lambda i, h: (0, 0, 0)),
            pl.BlockSpec((1, DH, D_MDL), lambda i, h: (h, 0, 0)),
            pl.BlockSpec((1, D_MDL), lambda i, h: (0, 0)),
        ],
        out_specs=pl.BlockSpec((_ROWS, D_MDL), lambda i, h: (i, 0)),
        out_shape=jax.ShapeDtypeStruct((L, D_MDL), jnp.float32),
    )(qh3, kh3, wmat, vaug, wo3, bo)


def kernel(queries, keys, values, adj, edge_pos_enc, Wq, bq, Wk, bk, Wv, bv,
           Wo, bo):
    q2 = queries.reshape(L, D_IN)
    k2 = keys.reshape(L, D_IN)
    v2 = values.reshape(L, D_IN)

    qh3, kh3, vaug = _project(q2, k2, v2, Wq, Wk, Wv,
                              bq.reshape(1, D_MDL),
                              bk.reshape(1, D_MDL),
                              bv.reshape(1, D_MDL))

    wmat = _build_w(adj[0], adj[1], edge_pos_enc)
    wo3 = Wo.astype(jnp.bfloat16).reshape(H, DH, D_MDL)

    out = _attention(qh3, kh3, wmat, vaug, wo3, bo.reshape(1, D_MDL))
    return out.reshape(1, L, D_MDL)


# speedup vs baseline: 1.2755x; 1.0067x over previous
import functools

import jax
import jax.numpy as jnp
from jax import lax
from jax.experimental import pallas as pl
from jax.experimental.pallas import tpu as pltpu
from jax.experimental.pallas import tpu_sc as plsc

L = 2048
D_IN = 1024
D_MDL = 1024
H = 16
DH = D_MDL // H
E = 65536
TEMP = 0.125


_ROWS = 256
_NT = L // _ROWS

def _split3(x):
    hi = x.astype(jnp.bfloat16)
    lo = (x - hi.astype(jnp.float32)).astype(jnp.bfloat16)
    return hi, lo


def _dot3(a, b, dims):
    ah, al = _split3(a)
    bh, bl = _split3(b)
    d = lambda x, y: lax.dot_general(x, y, dims,
                                     preferred_element_type=jnp.float32)
    return d(ah, bh) + d(ah, bl) + d(al, bh)


_MM_DIMS = (((1,), (0,)), ((), ()))
_QK_DIMS = (((1,), (1,)), ((), ()))


def _proj_body(xq, xk, xv, wq, wk, wv, bq, bk, bv, oqh, okh, ov):
    lq = (_dot3(xq[...], wq[...], _MM_DIMS) + bq[...]) * TEMP
    qh = lq.astype(jnp.bfloat16)
    lk = _dot3(xk[...], wk[...], _MM_DIMS) + bk[...]
    kh = lk.astype(jnp.bfloat16)
    vb = (_dot3(xv[...], wv[...], _MM_DIMS) + bv[...]).astype(jnp.bfloat16)
    zcol = jnp.zeros((_ROWS, 128 - DH - 1), jnp.bfloat16)
    onecol = jnp.ones((_ROWS, 1), jnp.bfloat16)
    for h in range(H):
        lo, hi = h * DH, (h + 1) * DH
        oqh[h] = qh[:, lo:hi]
        okh[h] = kh[:, lo:hi]
        ov[h] = jnp.concatenate([vb[:, lo:hi], onecol, zcol], axis=-1)


def _project(q2, k2, v2, Wq, Wk, Wv, bq, bk, bv):
    x_spec = pl.BlockSpec((_ROWS, D_IN), lambda i: (i, 0))
    w_spec = pl.BlockSpec((D_IN, D_MDL), lambda i: (0, 0))
    b_spec = pl.BlockSpec((1, D_MDL), lambda i: (0, 0))
    o_spec = pl.BlockSpec((H, _ROWS, DH), lambda i: (0, i, 0))
    o_sd = jax.ShapeDtypeStruct((H, L, DH), jnp.bfloat16)
    va_spec = pl.BlockSpec((H, _ROWS, 128), lambda i: (0, i, 0))
    va_sd = jax.ShapeDtypeStruct((H, L, 128), jnp.bfloat16)
    return pl.pallas_call(
        _proj_body,
        grid=(_NT,),
        in_specs=[x_spec, x_spec, x_spec, w_spec, w_spec, w_spec,
                  b_spec, b_spec, b_spec],
        out_specs=[o_spec, o_spec, va_spec],
        out_shape=[o_sd, o_sd, va_sd],
    )(q2, k2, v2, Wq, Wk, Wv, bq, bk, bv)


_NS = 16
_EPW = E // _NS
_NCHUNK = _EPW // 128
_QROWS = 512
_QWORDS = _QROWS * L
_DUMP = _QWORDS
_ZW = 8192
_SLICE = _QWORDS // _NS


def _build_w_body(a0_hbm, a1_hbm, epe_hbm, out_hbm,
                  a0_v, a1_v, val_v, idx_v, zeros_v, flush_v, w_sh, sem):
    c = lax.axis_index("c")
    s = lax.axis_index("s")
    base_e = s * _EPW

    cp0 = pltpu.async_copy(a0_hbm.at[pl.ds(base_e, _EPW)], a0_v, sem)
    cp1 = pltpu.async_copy(a1_hbm.at[pl.ds(base_e, _EPW)], a1_v, sem)
    cp2 = pltpu.async_copy(epe_hbm.at[pl.ds(base_e, _EPW)], val_v, sem)

    @plsc.parallel_loop(0, _ZW // 16, unroll=8)
    def _zero_step(t):
        zeros_v[pl.ds(t * 16, 16)] = jnp.zeros((16,), jnp.float32)

    cp0.wait()
    cp1.wait()
    cp2.wait()

    @plsc.parallel_loop(0, _EPW // 16, unroll=8)
    def _val_step(t):
        x = val_v[pl.ds(t * 16, 16)]
        val_v[pl.ds(t * 16, 16)] = jnp.exp(x * TEMP)

    for p in range(2):
        base_row = c * (2 * _QROWS) + p * _QROWS

        zcps = [pltpu.async_copy(
                    zeros_v, w_sh.at[pl.ds(s * _SLICE + z * _ZW, _ZW)], sem)
                for z in range(_SLICE // _ZW)]
        @plsc.parallel_loop(0, _EPW // 16, unroll=8)
        def _idx_step(t):
            a0 = a0_v[pl.ds(t * 16, 16)]
            a1 = a1_v[pl.ds(t * 16, 16)]
            rel = a0 - base_row
            ok = (rel >= 0) & (rel < _QROWS)
            idx_v[t // 8, pl.ds((t % 8) * 16, 16)] = jnp.where(
                ok, rel * L + a1, _DUMP)

        @plsc.parallel_loop(0, 8, unroll=8)
        def _dump_step(t):
            idx_v[_NCHUNK, pl.ds(t * 16, 16)] = jnp.full(
                (16,), _DUMP, jnp.int32)
        for cp in zcps:
            cp.wait()
        plsc.subcore_barrier()

        scps = [pltpu.async_copy(val_v.at[pl.ds(j * 128, 128)],
                                 w_sh.at[idx_v.at[j]], sem, add=True)
                for j in range(_NCHUNK)]
        scps += [pltpu.async_copy(zeros_v.at[pl.ds(0, 128)],
                                  w_sh.at[idx_v.at[_NCHUNK]], sem, add=True)
                 for _ in range(2)]
        for cp in scps:
            cp.wait()
        pltpu.sync_copy(zeros_v.at[pl.ds(0, 16)],
                        w_sh.at[pl.ds(_QWORDS, 16)])
        pltpu.sync_copy(w_sh.at[pl.ds(_QWORDS, 16)], flush_v)
        plsc.subcore_barrier()
        row0 = base_row + s * (_SLICE // L)
        ocps = [pltpu.async_copy(
                    w_sh.at[pl.ds(s * _SLICE + r * L, L)],
                    out_hbm.at[row0 + r], sem)
                for r in range(_SLICE // L)]
        for cp in ocps:
            cp.wait()
        plsc.subcore_barrier()


def _build_w(a0, a1, epe):
    mesh = plsc.VectorSubcoreMesh(core_axis_name="c", subcore_axis_name="s")
    return pl.kernel(
        _build_w_body,
        out_type=jax.ShapeDtypeStruct((L, L), jnp.float32),
        mesh=mesh,
        scratch_types=[
            pltpu.VMEM((_EPW,), jnp.int32),
            pltpu.VMEM((_EPW,), jnp.int32),
            pltpu.VMEM((_EPW,), jnp.float32),
            pltpu.VMEM((_NCHUNK + 1, 128), jnp.int32),
            pltpu.VMEM((_ZW,), jnp.float32),
            pltpu.VMEM((16,), jnp.float32),
            pltpu.VMEM_SHARED((_QWORDS + 16,), jnp.float32),
            pltpu.SemaphoreType.DMA,
        ],
    )(a0, a1, epe)


def _attn_body(qh, kh, w_ref, vaug, wo3, bo_ref, out_ref):
    h = pl.program_id(1)
    s = lax.dot_general(qh[0], kh[h], _QK_DIMS,
                        preferred_element_type=jnp.float32)
    p = (w_ref[...] * jnp.exp(s)).astype(jnp.bfloat16)
    pv = lax.dot_general(p, vaug[h], _MM_DIMS,
                         preferred_element_type=jnp.float32)
    den = pv[:, DH:DH + 1] + 1e-16
    ho = pv[:, :DH] / den
    contrib = lax.dot_general(ho.astype(jnp.bfloat16), wo3[0], _MM_DIMS,
                              preferred_element_type=jnp.float32)

    @pl.when(h == 0)
    def _():
        out_ref[...] = contrib + bo_ref[...]

    @pl.when(h > 0)
    def _():
        out_ref[...] += contrib


def _attention(qh3, kh3, wmat, vaug, wo3, bo):
    return pl.pallas_call(
        _attn_body,
        grid=(_NT, H),
        in_specs=[
            pl.BlockSpec((1, _ROWS, DH), lambda i, h: (h, i, 0)),
            pl.BlockSpec((H, L, DH), lambda i, h: (0, 0, 0)),
            pl.BlockSpec((_ROWS, L), lambda i, h: (i, 0)),
            pl.BlockSpec((H, L, 128), lambda i, h: (0, 0, 0)),
            pl.BlockSpec((1, DH, D_MDL), lambda i, h: (h, 0, 0)),
            pl.BlockSpec((1, D_MDL), lambda i, h: (0, 0)),
        ],
        out_specs=pl.BlockSpec((_ROWS, D_MDL), lambda i, h: (i, 0)),
        out_shape=jax.ShapeDtypeStruct((L, D_MDL), jnp.float32),
    )(qh3, kh3, wmat, vaug, wo3, bo)


def kernel(queries, keys, values, adj, edge_pos_enc, Wq, bq, Wk, bk, Wv, bv,
           Wo, bo):
    q2 = queries.reshape(L, D_IN)
    k2 = keys.reshape(L, D_IN)
    v2 = values.reshape(L, D_IN)

    qh3, kh3, vaug = _project(q2, k2, v2, Wq, Wk, Wv,
                              bq.reshape(1, D_MDL),
                              bk.reshape(1, D_MDL),
                              bv.reshape(1, D_MDL))

    wmat = _build_w(adj[0], adj[1], edge_pos_enc)
    wo3 = Wo.astype(jnp.bfloat16).reshape(H, DH, D_MDL)

    out = _attention(qh3, kh3, wmat, vaug, wo3, bo.reshape(1, D_MDL))
    return out.reshape(1, L, D_MDL)


# speedup vs baseline: 1.2790x; 1.0028x over previous
import functools

import jax
import jax.numpy as jnp
from jax import lax
from jax.experimental import pallas as pl
from jax.experimental.pallas import tpu as pltpu
from jax.experimental.pallas import tpu_sc as plsc

L = 2048
D_IN = 1024
D_MDL = 1024
H = 16
DH = D_MDL // H
E = 65536
TEMP = 0.125


_ROWS = 256
_NT = L // _ROWS

def _split3(x):
    hi = x.astype(jnp.bfloat16)
    lo = (x - hi.astype(jnp.float32)).astype(jnp.bfloat16)
    return hi, lo


def _dot3(a, b, dims):
    ah, al = _split3(a)
    bh, bl = _split3(b)
    d = lambda x, y: lax.dot_general(x, y, dims,
                                     preferred_element_type=jnp.float32)
    return d(ah, bh) + d(ah, bl) + d(al, bh)


_MM_DIMS = (((1,), (0,)), ((), ()))
_QK_DIMS = (((1,), (1,)), ((), ()))


def _proj_body(xq, xk, xv, wq, wk, wv, bq, bk, bv, oqh, okh, ov):
    lq = (_dot3(xq[...], wq[...], _MM_DIMS) + bq[...]) * TEMP
    qh = lq.astype(jnp.bfloat16)
    lk = _dot3(xk[...], wk[...], _MM_DIMS) + bk[...]
    kh = lk.astype(jnp.bfloat16)
    vb = (_dot3(xv[...], wv[...], _MM_DIMS) + bv[...]).astype(jnp.bfloat16)
    zcol = jnp.zeros((_ROWS, 128 - DH - 1), jnp.bfloat16)
    onecol = jnp.ones((_ROWS, 1), jnp.bfloat16)
    for h in range(H):
        lo, hi = h * DH, (h + 1) * DH
        oqh[h] = qh[:, lo:hi]
        okh[h] = kh[:, lo:hi]
        ov[h] = jnp.concatenate([vb[:, lo:hi], onecol, zcol], axis=-1)


def _project(q2, k2, v2, Wq, Wk, Wv, bq, bk, bv):
    x_spec = pl.BlockSpec((_ROWS, D_IN), lambda i: (i, 0))
    w_spec = pl.BlockSpec((D_IN, D_MDL), lambda i: (0, 0))
    b_spec = pl.BlockSpec((1, D_MDL), lambda i: (0, 0))
    o_spec = pl.BlockSpec((H, _ROWS, DH), lambda i: (0, i, 0))
    o_sd = jax.ShapeDtypeStruct((H, L, DH), jnp.bfloat16)
    va_spec = pl.BlockSpec((H, _ROWS, 128), lambda i: (0, i, 0))
    va_sd = jax.ShapeDtypeStruct((H, L, 128), jnp.bfloat16)
    return pl.pallas_call(
        _proj_body,
        grid=(_NT,),
        in_specs=[x_spec, x_spec, x_spec, w_spec, w_spec, w_spec,
                  b_spec, b_spec, b_spec],
        out_specs=[o_spec, o_spec, va_spec],
        out_shape=[o_sd, o_sd, va_sd],
    )(q2, k2, v2, Wq, Wk, Wv, bq, bk, bv)


_NS = 16
_EPW = E // _NS
_NCHUNK = _EPW // 128
_QROWS = 512
_QWORDS = _QROWS * L
_DUMP = _QWORDS
_ZW = 8192
_SLICE = _QWORDS // _NS


def _build_w_body(a0_hbm, a1_hbm, epe_hbm, out_hbm,
                  a0_v, a1_v, val_v, idx2_v, dump_v, zeros_v, flush_v,
                  w_sh, sem):
    c = lax.axis_index("c")
    s = lax.axis_index("s")
    base_e = s * _EPW
    cp0 = pltpu.async_copy(a0_hbm.at[pl.ds(base_e, _EPW)], a0_v, sem)
    cp1 = pltpu.async_copy(a1_hbm.at[pl.ds(base_e, _EPW)], a1_v, sem)
    cp2 = pltpu.async_copy(epe_hbm.at[pl.ds(base_e, _EPW)], val_v, sem)
    cp0.wait()
    cp1.wait()
    cp2.wait()

    @plsc.parallel_loop(0, _EPW // 16, unroll=8)
    def _val_step(t):
        x = val_v[pl.ds(t * 16, 16)]
        val_v[pl.ds(t * 16, 16)] = jnp.exp(x * TEMP)

    @plsc.parallel_loop(0, _ZW // 16, unroll=8)
    def _zero_step(t):
        zeros_v[pl.ds(t * 16, 16)] = jnp.zeros((16,), jnp.float32)

    @plsc.parallel_loop(0, 8, unroll=8)
    def _dump_step(t):
        dump_v[pl.ds(t * 16, 16)] = jnp.full((16,), _DUMP, jnp.int32)

    for p in range(2):
        base_row = c * (2 * _QROWS) + p * _QROWS
        zcps = [pltpu.async_copy(
                    zeros_v, w_sh.at[pl.ds(s * _SLICE + z * _ZW, _ZW)], sem)
                for z in range(_SLICE // _ZW)]
        @plsc.parallel_loop(0, _EPW // 16, unroll=8)
        def _idx_step(t):
            a0 = a0_v[pl.ds(t * 16, 16)]
            a1 = a1_v[pl.ds(t * 16, 16)]
            rel = a0 - base_row
            ok = (rel >= 0) & (rel < _QROWS)
            idx2_v[pl.ds(t * 16, 16)] = jnp.where(ok, rel * L + a1, _DUMP)

        for cp in zcps:
            cp.wait()
        plsc.subcore_barrier()
        cps = [pltpu.async_copy(val_v, w_sh.at[idx2_v], sem, add=True)]
        cps += [pltpu.async_copy(zeros_v.at[pl.ds(0, 128)],
                                 w_sh.at[dump_v], sem, add=True)
                for _ in range(2)]
        for cp in cps:
            cp.wait()
        pltpu.sync_copy(zeros_v.at[pl.ds(0, 16)],
                        w_sh.at[pl.ds(_QWORDS, 16)])
        pltpu.sync_copy(w_sh.at[pl.ds(_QWORDS, 16)], flush_v)
        plsc.subcore_barrier()
        row0 = base_row + s * (_SLICE // L)
        ocps = [pltpu.async_copy(
                    w_sh.at[pl.ds(s * _SLICE + r * L, L)],
                    out_hbm.at[row0 + r], sem)
                for r in range(_SLICE // L)]
        for cp in ocps:
            cp.wait()
        plsc.subcore_barrier()


def _build_w(a0, a1, epe):
    mesh = plsc.VectorSubcoreMesh(core_axis_name="c", subcore_axis_name="s")
    return pl.kernel(
        _build_w_body,
        out_type=jax.ShapeDtypeStruct((L, L), jnp.float32),
        mesh=mesh,
        scratch_types=[
            pltpu.VMEM((_EPW,), jnp.int32),
            pltpu.VMEM((_EPW,), jnp.int32),
            pltpu.VMEM((_EPW,), jnp.float32),
            pltpu.VMEM((_EPW,), jnp.int32),
            pltpu.VMEM((128,), jnp.int32),
            pltpu.VMEM((_ZW,), jnp.float32),
            pltpu.VMEM((16,), jnp.float32),
            pltpu.VMEM_SHARED((_QWORDS + 16,), jnp.float32),
            pltpu.SemaphoreType.DMA,
        ],
    )(a0, a1, epe)


def _attn_body(qh, kh, w_ref, vaug, wo3, bo_ref, out_ref):
    h = pl.program_id(1)
    s = lax.dot_general(qh[0], kh[h], _QK_DIMS,
                        preferred_element_type=jnp.float32)
    p = (w_ref[...] * jnp.exp(s)).astype(jnp.bfloat16)
    pv = lax.dot_general(p, vaug[h], _MM_DIMS,
                         preferred_element_type=jnp.float32)
    den = pv[:, DH:DH + 1] + 1e-16
    ho = pv[:, :DH] / den
    contrib = lax.dot_general(ho.astype(jnp.bfloat16), wo3[0], _MM_DIMS,
                              preferred_element_type=jnp.float32)

    @pl.when(h == 0)
    def _():
        out_ref[...] = contrib + bo_ref[...]

    @pl.when(h > 0)
    def _():
        out_ref[...] += contrib


def _attention(qh3, kh3, wmat, vaug, wo3, bo):
    return pl.pallas_call(
        _attn_body,
        grid=(_NT, H),
        in_specs=[
            pl.BlockSpec((1, _ROWS, DH), lambda i, h: (h, i, 0)),
            pl.BlockSpec((H, L, DH), lambda i, h: (0, 0, 0)),
            pl.BlockSpec((_ROWS, L), lambda i, h: (i, 0)),
            pl.BlockSpec((H, L, 128), lambda i, h: (0, 0, 0)),
            pl.BlockSpec((1, DH, D_MDL), lambda i, h: (h, 0, 0)),
            pl.BlockSpec((1, D_MDL), lambda i, h: (0, 0)),
        ],
        out_specs=pl.BlockSpec((_ROWS, D_MDL), lambda i, h: (i, 0)),
        out_shape=jax.ShapeDtypeStruct((L, D_MDL), jnp.float32),
    )(qh3, kh3, wmat, vaug, wo3, bo)


def kernel(queries, keys, values, adj, edge_pos_enc, Wq, bq, Wk, bk, Wv, bv,
           Wo, bo):
    q2 = queries.reshape(L, D_IN)
    k2 = keys.reshape(L, D_IN)
    v2 = values.reshape(L, D_IN)

    qh3, kh3, vaug = _project(q2, k2, v2, Wq, Wk, Wv,
                              bq.reshape(1, D_MDL),
                              bk.reshape(1, D_MDL),
                              bv.reshape(1, D_MDL))

    wmat = _build_w(adj[0], adj[1], edge_pos_enc)
    wo3 = Wo.astype(jnp.bfloat16).reshape(H, DH, D_MDL)

    out = _attention(qh3, kh3, wmat, vaug, wo3, bo.reshape(1, D_MDL))
    return out.reshape(1, L, D_MDL)
